# baseline (speedup 1.0000x reference)
import functools

import jax
import jax.numpy as jnp
from jax import lax
from jax.experimental import pallas as pl
from jax.experimental.pallas import tpu as pltpu
from jax.experimental.pallas import tpu_sc as plsc

_N = 100000
_NP = 100352
_W = 32
_CH = _NP // _W
_STEPS = _CH // 16
_K = 256
_TILE = 4000
_T = _N // _TILE
_QMIN = 0.5
_SB = 1.0
_BIG = float(_N)


def _q_of(beta):
    return (0.5 * jnp.log((1.0 + beta) / (1.0 - beta))) ** 2 + _QMIN


def _sc_a_body(tid_h, beta_h, pe_h, ppx_h, ppy_h, pt_h, te_h, tpx_h, tpy_h,
               tt_h, stats_h, noise_h, tid_v, b_v, pe_v, ppx_v, ppy_v, pt_v,
               te_v, tpx_v, tpy_v, tt_v, segt, bkt, pnt, pdt, stage, nstage):
    c = lax.axis_index("c")
    s = lax.axis_index("s")
    w = s * 2 + c
    base = w * _CH
    pltpu.sync_copy(tid_h.at[pl.ds(base, _CH)], tid_v)
    pltpu.sync_copy(beta_h.at[pl.ds(base, _CH)], b_v)
    pltpu.sync_copy(pe_h.at[pl.ds(base, _CH)], pe_v)
    pltpu.sync_copy(ppx_h.at[pl.ds(base, _CH)], ppx_v)
    pltpu.sync_copy(ppy_h.at[pl.ds(base, _CH)], ppy_v)
    pltpu.sync_copy(pt_h.at[pl.ds(base, _CH)], pt_v)
    pltpu.sync_copy(te_h.at[pl.ds(base, _CH)], te_v)
    pltpu.sync_copy(tpx_h.at[pl.ds(base, _CH)], tpx_v)
    pltpu.sync_copy(tpy_h.at[pl.ds(base, _CH)], tpy_v)
    pltpu.sync_copy(tt_h.at[pl.ds(base, _CH)], tt_v)

    def initloop(j, _):
        sl = pl.ds(j * 16, 16)
        segt[sl] = jnp.zeros((16,), jnp.float32)
        bkt[sl] = jnp.full((16,), -jnp.inf, jnp.float32)
        pnt[sl] = jnp.zeros((16,), jnp.float32)
        pdt[sl] = jnp.zeros((16,), jnp.float32)
        return 0

    lax.fori_loop(0, 256, initloop, 0)

    lane = lax.iota(jnp.int32, 16)
    lanek = lane * _K

    def body(j, carry):
        nb, nn = carry
        sl = pl.ds(j * 16, 16)
        tid16 = tid_v[sl]
        b16 = jnp.clip(b_v[sl], 1e-6, 1.0 - 1e-6)
        noise = tid16 == 0
        pidx = lanek + tid16
        objw = jnp.where(noise, 0.0, 1.0)
        bmask = jnp.where(noise, -1.0, b16)
        te16 = te_v[sl]
        ew = jnp.maximum(
            jnp.where(te16 > 10.0, 1.0, (te16 - 0.5) / 10.0 * 10.0 / 9.5),
            0.0)
        den = te16 + 1.0
        denz = den == 0.0
        de = te16 - pe_v[sl]
        dx = tpx_v[sl] - ppx_v[sl]
        dy = tpy_v[sl] - ppy_v[sl]
        dt16 = tt_v[sl] - pt_v[sl]
        le = jnp.where(denz, 0.0, de * de / jnp.where(denz, 1.0, den))
        lpos = (dx * dx + dy * dy) / 100.0
        lt = dt16 * dt16
        pay = ew * le + lpos + lt
        pw = jnp.where(noise, 0.0, b16)
        old = plsc.load_gather(segt, [pidx])
        plsc.store_scatter(segt, [pidx], old + objw)
        oldb = plsc.load_gather(bkt, [pidx])
        plsc.store_scatter(bkt, [pidx], jnp.maximum(oldb, bmask))
        oldn = plsc.load_gather(pnt, [pidx])
        plsc.store_scatter(pnt, [pidx], oldn + pw * pay)
        oldd = plsc.load_gather(pdt, [pidx])
        plsc.store_scatter(pdt, [pidx], oldd + pw)
        gi = base + j * 16 + lane
        realn = (gi < _N) & noise
        nb = nb + jnp.where(realn, b16, 0.0)
        nn = nn + jnp.where(realn, 1.0, 0.0)
        return nb, nn

    nb, nn = lax.fori_loop(
        0, _STEPS, body,
        (jnp.zeros((16,), jnp.float32), jnp.zeros((16,), jnp.float32)))

    def fold(v, _):
        sl = pl.ds(v * 16, 16)
        a0 = segt[sl]
        a1 = bkt[sl]
        a2 = pnt[sl]
        a3 = pdt[sl]
        for r in range(1, 16):
            slr = pl.ds(r * _K + v * 16, 16)
            a0 = a0 + segt[slr]
            a1 = jnp.maximum(a1, bkt[slr])
            a2 = a2 + pnt[slr]
            a3 = a3 + pdt[slr]
        stage[0, sl] = a0
        stage[1, sl] = a1
        stage[2, sl] = a2
        stage[3, sl] = a3
        return 0

    lax.fori_loop(0, 16, fold, 0)
    nstage[pl.ds(0, 16)] = nb
    nstage[pl.ds(16, 16)] = nn

    def nz(j, _):
        nstage[pl.ds(32 + j * 16, 16)] = jnp.zeros((16,), jnp.float32)
        return 0

    lax.fori_loop(0, 6, nz, 0)
    pltpu.sync_copy(stage, stats_h.at[w])
    pltpu.sync_copy(nstage, noise_h.at[w])


_sc_a = pl.kernel(
    _sc_a_body,
    out_type=(jax.ShapeDtypeStruct((_W, 4, _K), jnp.float32),
              jax.ShapeDtypeStruct((_W, 128), jnp.float32)),
    mesh=plsc.VectorSubcoreMesh(core_axis_name="c", subcore_axis_name="s",
                                num_cores=2, num_subcores=16),
    scratch_types=[pltpu.VMEM((_CH,), jnp.int32)]
    + [pltpu.VMEM((_CH,), jnp.float32)] * 9
    + [pltpu.VMEM((16 * _K,), jnp.float32)] * 4
    + [pltpu.VMEM((4, _K), jnp.float32), pltpu.VMEM((128,), jnp.float32)],
    compiler_params=pltpu.CompilerParams(needs_layout_passes=False),
)


def _sc_b_body(tid_h, beta_h, x0_h, x1_h, stats_h, bpart_h, tid_v, b_v, x0_v,
               x1_v, stats_v, bk256, amt, ax0t, ax1t, abt, stage):
    c = lax.axis_index("c")
    s = lax.axis_index("s")
    w = s * 2 + c
    base = w * _CH
    pltpu.sync_copy(tid_h.at[pl.ds(base, _CH)], tid_v)
    pltpu.sync_copy(beta_h.at[pl.ds(base, _CH)], b_v)
    pltpu.sync_copy(x0_h.at[pl.ds(base, _CH)], x0_v)
    pltpu.sync_copy(x1_h.at[pl.ds(base, _CH)], x1_v)
    pltpu.sync_copy(stats_h, stats_v)

    def redk(v, _):
        sl = pl.ds(v * 16, 16)
        acc = stats_v[0, 1, sl]
        for i in range(1, _W):
            acc = jnp.maximum(acc, stats_v[i, 1, sl])
        bk256[sl] = acc
        return 0

    lax.fori_loop(0, 16, redk, 0)

    def initloop(j, _):
        sl = pl.ds(j * 16, 16)
        amt[sl] = jnp.full((16,), _BIG, jnp.float32)
        ax0t[sl] = jnp.zeros((16,), jnp.float32)
        ax1t[sl] = jnp.zeros((16,), jnp.float32)
        abt[sl] = jnp.zeros((16,), jnp.float32)
        return 0

    lax.fori_loop(0, 256, initloop, 0)

    lane = lax.iota(jnp.int32, 16)
    lanek = lane * _K

    def body(j, _):
        sl = pl.ds(j * 16, 16)
        tid16 = tid_v[sl]
        b16 = jnp.clip(b_v[sl], 1e-6, 1.0 - 1e-6)
        noise = tid16 == 0
        bmask = jnp.where(noise, -1.0, b16)
        bk16 = plsc.load_gather(bk256, [tid16])
        gi = base + j * 16 + lane
        isal = (bmask == bk16) & (~noise)
        candf = jnp.where(isal, gi.astype(jnp.float32), _BIG)
        pidx = lanek + tid16
        old = plsc.load_gather(amt, [pidx])
        upd = candf < old
        plsc.store_scatter(amt, [pidx], jnp.where(upd, candf, old))
        plsc.store_scatter(ax0t, [pidx], x0_v[sl], mask=upd)
        plsc.store_scatter(ax1t, [pidx], x1_v[sl], mask=upd)
        plsc.store_scatter(abt, [pidx], b16, mask=upd)
        return 0

    lax.fori_loop(0, _STEPS, body, 0)

    def fold(v, _):
        sl = pl.ds(v * 16, 16)
        am = amt[sl]
        a0 = ax0t[sl]
        a1 = ax1t[sl]
        ab = abt[sl]
        for r in range(1, 16):
            slr = pl.ds(r * _K + v * 16, 16)
            m = amt[slr]
            u = m < am
            am = jnp.where(u, m, am)
            a0 = jnp.where(u, ax0t[slr], a0)
            a1 = jnp.where(u, ax1t[slr], a1)
            ab = jnp.where(u, abt[slr], ab)
        stage[0, sl] = am
        stage[1, sl] = a0
        stage[2, sl] = a1
        stage[3, sl] = ab
        return 0

    lax.fori_loop(0, 16, fold, 0)
    pltpu.sync_copy(stage, bpart_h.at[w])


_sc_b = pl.kernel(
    _sc_b_body,
    out_type=jax.ShapeDtypeStruct((_W, 4, _K), jnp.float32),
    mesh=plsc.VectorSubcoreMesh(core_axis_name="c", subcore_axis_name="s",
                                num_cores=2, num_subcores=16),
    scratch_types=[pltpu.VMEM((_CH,), jnp.int32)]
    + [pltpu.VMEM((_CH,), jnp.float32)] * 3
    + [pltpu.VMEM((_W, 4, _K), jnp.float32),
       pltpu.VMEM((_K,), jnp.float32)]
    + [pltpu.VMEM((16 * _K,), jnp.float32)] * 4
    + [pltpu.VMEM((4, _K), jnp.float32)],
    compiler_params=pltpu.CompilerParams(needs_layout_passes=False),
)


def _tc_q_body(b_ref, q_ref):
    beta = jnp.clip(b_ref[...], 1e-6, 1.0 - 1e-6)
    q_ref[...] = _q_of(beta)


@jax.jit
def _tc_q(beta2d):
    return pl.pallas_call(
        _tc_q_body,
        out_shape=jax.ShapeDtypeStruct(beta2d.shape, jnp.float32),
    )(beta2d)


def _tc_c_body(feat_ref, stats_ref, bpart_ref, noise_ref, out_ref, xa0r, xa1r,
               wrow, smem):
    t = pl.program_id(0)
    iotak = jax.lax.broadcasted_iota(jnp.int32, (1, _K), 1)

    @pl.when(t == 0)
    def _pro():
        s3 = stats_ref[...]
        segc = jnp.sum(s3[:, 0, :], axis=0, keepdims=True)
        betak = jnp.max(s3[:, 1, :], axis=0, keepdims=True)
        paynum = jnp.sum(s3[:, 2, :], axis=0, keepdims=True)
        payden = jnp.sum(s3[:, 3, :], axis=0, keepdims=True)
        b3 = bpart_ref[...]
        am = b3[:, 0, :]
        tm = jnp.min(am, axis=0, keepdims=True)
        sel = (am == tm) & (am < _BIG)
        xa0 = jnp.sum(jnp.where(sel, b3[:, 1, :], 0.0), axis=0, keepdims=True)
        xa1 = jnp.sum(jnp.where(sel, b3[:, 2, :], 0.0), axis=0, keepdims=True)
        ab = jnp.sum(jnp.where(sel, b3[:, 3, :], 0.0), axis=0, keepdims=True)
        validf = ((segc > 0.0) & (iotak > 0)).astype(jnp.float32)
        ba = jnp.clip(ab, 1e-6, 1.0 - 1e-6)
        qa = _q_of(ba)
        xa0r[...] = xa0
        xa1r[...] = xa1
        wrow[...] = qa * validf
        nv = jnp.sum(validf)
        nrow = jnp.sum(noise_ref[...], axis=0, keepdims=True)
        nbsum = jnp.sum(nrow[:, 0:16])
        nnsum = jnp.sum(nrow[:, 16:32])
        noise_l = _SB * nbsum / (nnsum + 1e-9)
        bkv = jnp.clip(betak, 0.0, 1.0)
        beta_obj = jnp.sum(jnp.where(validf > 0.0, 1.0 - bkv, 0.0)) / (
            nv + 1e-9)
        pdz = payden == 0.0
        payk = jnp.where(pdz, 0.0, paynum / jnp.where(pdz, 1.0, payden))
        pay_l = jnp.sum(validf * payk) / (nv + 1e-9)
        smem[0] = 0.0
        smem[1] = 0.0
        smem[2] = noise_l + beta_obj + pay_l
        smem[3] = jnp.sum(segc * validf) + 1e-9
        smem[4] = jnp.float32(_N) * nv + 1e-9
        out_ref[...] = jnp.zeros_like(out_ref)

    feat = feat_ref[...]
    q = feat[:, 0:1]
    x0 = feat[:, 1:2]
    x1 = feat[:, 2:3]
    tidf = feat[:, 3:4]
    onehot = tidf == iotak.astype(jnp.float32)
    d2 = (x0 - xa0r[...]) ** 2 + (x1 - xa1r[...]) ** 2 + 1e-6
    dist = d2 * lax.rsqrt(d2)
    r = jnp.maximum(1.0 - dist, 0.0)
    rq = r * q
    d2q = d2 * q
    attcol = jnp.sum(jnp.where(onehot, d2q, 0.0), axis=0, keepdims=True)
    repcol = jnp.sum(jnp.where(onehot, 0.0, rq), axis=0, keepdims=True)
    smem[0] += jnp.sum(attcol * wrow[...])
    smem[1] += jnp.sum(repcol * wrow[...])

    @pl.when(t == _T - 1)
    def _fin():
        loss = smem[0] / smem[3] + smem[1] / smem[4] + smem[2]
        out_ref[...] = loss * jnp.ones_like(out_ref)


@jax.jit
def _tc_c(feat, stats, bpart, noisep):
    return pl.pallas_call(
        _tc_c_body,
        grid=(_T,),
        in_specs=[
            pl.BlockSpec((_TILE, 4), lambda t: (t, 0)),
            pl.BlockSpec((_W, 4, _K), lambda t: (0, 0, 0)),
            pl.BlockSpec((_W, 4, _K), lambda t: (0, 0, 0)),
            pl.BlockSpec((_W, 128), lambda t: (0, 0)),
        ],
        out_specs=pl.BlockSpec((1, 1), lambda t: (0, 0)),
        out_shape=jax.ShapeDtypeStruct((1, 1), jnp.float32),
        scratch_shapes=[
            pltpu.VMEM((1, _K), jnp.float32),
            pltpu.VMEM((1, _K), jnp.float32),
            pltpu.VMEM((1, _K), jnp.float32),
            pltpu.SMEM((8,), jnp.float32),
        ],
    )(feat, stats, bpart, noisep)


def _pad(a):
    return jnp.concatenate([a, jnp.zeros((_NP - _N,), a.dtype)])


def kernel(pred_beta, pred_ccoords, pred_energy, pred_pos, pred_time,
           pred_id, t_idx, t_energy, t_pos, t_time):
    tid_p = _pad(t_idx.reshape(-1).astype(jnp.int32))
    beta_p = _pad(pred_beta[:, 0])
    x0_p = _pad(pred_ccoords[:, 0])
    x1_p = _pad(pred_ccoords[:, 1])
    pe_p = _pad(pred_energy[:, 0])
    ppx_p = _pad(pred_pos[:, 0])
    ppy_p = _pad(pred_pos[:, 1])
    pt_p = _pad(pred_time[:, 0])
    te_p = _pad(t_energy[:, 0])
    tpx_p = _pad(t_pos[:, 0])
    tpy_p = _pad(t_pos[:, 1])
    tt_p = _pad(t_time[:, 0])

    stats, noisep = _sc_a(tid_p, beta_p, pe_p, ppx_p, ppy_p, pt_p, te_p,
                          tpx_p, tpy_p, tt_p)
    bpart = _sc_b(tid_p, beta_p, x0_p, x1_p, stats)

    q2d = _tc_q(beta_p.reshape(_NP // 128, 128))
    q_col = q2d.reshape(-1)[:_N, None]
    tidf = t_idx.reshape(-1, 1).astype(jnp.float32)
    feat = jnp.concatenate([q_col, pred_ccoords, tidf], axis=1)
    loss = _tc_c(feat, stats, bpart, noisep)
    return (pred_beta, loss.reshape(1))


# speedup vs baseline: 1.0123x; 1.0123x over previous
import functools

import jax
import jax.numpy as jnp
from jax import lax
from jax.experimental import pallas as pl
from jax.experimental.pallas import tpu as pltpu
from jax.experimental.pallas import tpu_sc as plsc

_N = 100000
_W = 32
_CH = 3136
_CHL = _N - (_W - 1) * _CH
_STEPS = _CH // 16
_STEPSL = _CHL // 16
_K = 256
_TILE = 4000
_T = _N // _TILE
_QMIN = 0.5
_SB = 1.0
_BIG = float(_N)


def _q_of(beta):
    return (0.5 * jnp.log((1.0 + beta) / (1.0 - beta))) ** 2 + _QMIN


def _sc_a_body(tid_h, beta_h, pe_h, ppx_h, ppy_h, pt_h, te_h, tpx_h, tpy_h,
               tt_h, stats_h, noise_h, tid_v, b_v, pe_v, ppx_v, ppy_v, pt_v,
               te_v, tpx_v, tpy_v, tt_v, segt, bkt, pnt, pdt, stage, nstage):
    c = lax.axis_index("c")
    s = lax.axis_index("s")
    w = s * 2 + c
    base = w * _CH
    nsteps = jnp.where(w == _W - 1, _STEPSL, _STEPS)

    @pl.when(w < _W - 1)
    def _dma_full():
        pltpu.sync_copy(tid_h.at[pl.ds(base, _CH)], tid_v)
        pltpu.sync_copy(beta_h.at[pl.ds(base, _CH)], b_v)
        pltpu.sync_copy(pe_h.at[pl.ds(base, _CH)], pe_v)
        pltpu.sync_copy(ppx_h.at[pl.ds(base, _CH)], ppx_v)
        pltpu.sync_copy(ppy_h.at[pl.ds(base, _CH)], ppy_v)
        pltpu.sync_copy(pt_h.at[pl.ds(base, _CH)], pt_v)
        pltpu.sync_copy(te_h.at[pl.ds(base, _CH)], te_v)
        pltpu.sync_copy(tpx_h.at[pl.ds(base, _CH)], tpx_v)
        pltpu.sync_copy(tpy_h.at[pl.ds(base, _CH)], tpy_v)
        pltpu.sync_copy(tt_h.at[pl.ds(base, _CH)], tt_v)

    @pl.when(w == _W - 1)
    def _dma_last():
        dsl = pl.ds(0, _CHL)
        pltpu.sync_copy(tid_h.at[pl.ds(base, _CHL)], tid_v.at[dsl])
        pltpu.sync_copy(beta_h.at[pl.ds(base, _CHL)], b_v.at[dsl])
        pltpu.sync_copy(pe_h.at[pl.ds(base, _CHL)], pe_v.at[dsl])
        pltpu.sync_copy(ppx_h.at[pl.ds(base, _CHL)], ppx_v.at[dsl])
        pltpu.sync_copy(ppy_h.at[pl.ds(base, _CHL)], ppy_v.at[dsl])
        pltpu.sync_copy(pt_h.at[pl.ds(base, _CHL)], pt_v.at[dsl])
        pltpu.sync_copy(te_h.at[pl.ds(base, _CHL)], te_v.at[dsl])
        pltpu.sync_copy(tpx_h.at[pl.ds(base, _CHL)], tpx_v.at[dsl])
        pltpu.sync_copy(tpy_h.at[pl.ds(base, _CHL)], tpy_v.at[dsl])
        pltpu.sync_copy(tt_h.at[pl.ds(base, _CHL)], tt_v.at[dsl])

    def initloop(j, _):
        sl = pl.ds(j * 16, 16)
        segt[sl] = jnp.zeros((16,), jnp.float32)
        bkt[sl] = jnp.full((16,), -jnp.inf, jnp.float32)
        pnt[sl] = jnp.zeros((16,), jnp.float32)
        pdt[sl] = jnp.zeros((16,), jnp.float32)
        return 0

    lax.fori_loop(0, 256, initloop, 0)

    lane = lax.iota(jnp.int32, 16)
    lanek = lane * _K

    def body(j, carry):
        nb, nn = carry
        sl = pl.ds(j * 16, 16)
        tid16 = tid_v[sl]
        b16 = jnp.clip(b_v[sl], 1e-6, 1.0 - 1e-6)
        noise = tid16 == 0
        pidx = lanek + tid16
        objw = jnp.where(noise, 0.0, 1.0)
        bmask = jnp.where(noise, -1.0, b16)
        te16 = te_v[sl]
        ew = jnp.maximum(
            jnp.where(te16 > 10.0, 1.0, (te16 - 0.5) / 10.0 * 10.0 / 9.5),
            0.0)
        den = te16 + 1.0
        denz = den == 0.0
        de = te16 - pe_v[sl]
        dx = tpx_v[sl] - ppx_v[sl]
        dy = tpy_v[sl] - ppy_v[sl]
        dt16 = tt_v[sl] - pt_v[sl]
        le = jnp.where(denz, 0.0, de * de / jnp.where(denz, 1.0, den))
        lpos = (dx * dx + dy * dy) / 100.0
        lt = dt16 * dt16
        pay = ew * le + lpos + lt
        pw = jnp.where(noise, 0.0, b16)
        old = plsc.load_gather(segt, [pidx])
        plsc.store_scatter(segt, [pidx], old + objw)
        oldb = plsc.load_gather(bkt, [pidx])
        plsc.store_scatter(bkt, [pidx], jnp.maximum(oldb, bmask))
        oldn = plsc.load_gather(pnt, [pidx])
        plsc.store_scatter(pnt, [pidx], oldn + pw * pay)
        oldd = plsc.load_gather(pdt, [pidx])
        plsc.store_scatter(pdt, [pidx], oldd + pw)
        nb = nb + jnp.where(noise, b16, 0.0)
        nn = nn + jnp.where(noise, 1.0, 0.0)
        return nb, nn

    nb, nn = lax.fori_loop(
        0, nsteps, body,
        (jnp.zeros((16,), jnp.float32), jnp.zeros((16,), jnp.float32)))

    def fold(v, _):
        sl = pl.ds(v * 16, 16)
        a0 = segt[sl]
        a1 = bkt[sl]
        a2 = pnt[sl]
        a3 = pdt[sl]
        for r in range(1, 16):
            slr = pl.ds(r * _K + v * 16, 16)
            a0 = a0 + segt[slr]
            a1 = jnp.maximum(a1, bkt[slr])
            a2 = a2 + pnt[slr]
            a3 = a3 + pdt[slr]
        stage[0, sl] = a0
        stage[1, sl] = a1
        stage[2, sl] = a2
        stage[3, sl] = a3
        return 0

    lax.fori_loop(0, 16, fold, 0)
    nstage[pl.ds(0, 16)] = nb
    nstage[pl.ds(16, 16)] = nn

    def nz(j, _):
        nstage[pl.ds(32 + j * 16, 16)] = jnp.zeros((16,), jnp.float32)
        return 0

    lax.fori_loop(0, 6, nz, 0)
    pltpu.sync_copy(stage, stats_h.at[w])
    pltpu.sync_copy(nstage, noise_h.at[w])


_sc_a = pl.kernel(
    _sc_a_body,
    out_type=(jax.ShapeDtypeStruct((_W, 4, _K), jnp.float32),
              jax.ShapeDtypeStruct((_W, 128), jnp.float32)),
    mesh=plsc.VectorSubcoreMesh(core_axis_name="c", subcore_axis_name="s",
                                num_cores=2, num_subcores=16),
    scratch_types=[pltpu.VMEM((_CH,), jnp.int32)]
    + [pltpu.VMEM((_CH,), jnp.float32)] * 9
    + [pltpu.VMEM((16 * _K,), jnp.float32)] * 4
    + [pltpu.VMEM((4, _K), jnp.float32), pltpu.VMEM((128,), jnp.float32)],
    compiler_params=pltpu.CompilerParams(needs_layout_passes=False),
)


def _sc_b_body(tid_h, beta_h, x0_h, x1_h, stats_h, bpart_h, tid_v, b_v, x0_v,
               x1_v, stats_v, bk256, amt, ax0t, ax1t, abt, stage):
    c = lax.axis_index("c")
    s = lax.axis_index("s")
    w = s * 2 + c
    base = w * _CH
    nsteps = jnp.where(w == _W - 1, _STEPSL, _STEPS)

    @pl.when(w < _W - 1)
    def _dma_full():
        pltpu.sync_copy(tid_h.at[pl.ds(base, _CH)], tid_v)
        pltpu.sync_copy(beta_h.at[pl.ds(base, _CH)], b_v)
        pltpu.sync_copy(x0_h.at[pl.ds(base, _CH)], x0_v)
        pltpu.sync_copy(x1_h.at[pl.ds(base, _CH)], x1_v)

    @pl.when(w == _W - 1)
    def _dma_last():
        dsl = pl.ds(0, _CHL)
        pltpu.sync_copy(tid_h.at[pl.ds(base, _CHL)], tid_v.at[dsl])
        pltpu.sync_copy(beta_h.at[pl.ds(base, _CHL)], b_v.at[dsl])
        pltpu.sync_copy(x0_h.at[pl.ds(base, _CHL)], x0_v.at[dsl])
        pltpu.sync_copy(x1_h.at[pl.ds(base, _CHL)], x1_v.at[dsl])

    pltpu.sync_copy(stats_h, stats_v)

    def redk(v, _):
        sl = pl.ds(v * 16, 16)
        acc = stats_v[0, 1, sl]
        for i in range(1, _W):
            acc = jnp.maximum(acc, stats_v[i, 1, sl])
        bk256[sl] = acc
        return 0

    lax.fori_loop(0, 16, redk, 0)

    def initloop(j, _):
        sl = pl.ds(j * 16, 16)
        amt[sl] = jnp.full((16,), _BIG, jnp.float32)
        ax0t[sl] = jnp.zeros((16,), jnp.float32)
        ax1t[sl] = jnp.zeros((16,), jnp.float32)
        abt[sl] = jnp.zeros((16,), jnp.float32)
        return 0

    lax.fori_loop(0, 256, initloop, 0)

    lane = lax.iota(jnp.int32, 16)
    lanek = lane * _K

    def body(j, _):
        sl = pl.ds(j * 16, 16)
        tid16 = tid_v[sl]
        b16 = jnp.clip(b_v[sl], 1e-6, 1.0 - 1e-6)
        noise = tid16 == 0
        bmask = jnp.where(noise, -1.0, b16)
        bk16 = plsc.load_gather(bk256, [tid16])
        gi = base + j * 16 + lane
        isal = (bmask == bk16) & (~noise)
        candf = jnp.where(isal, gi.astype(jnp.float32), _BIG)
        pidx = lanek + tid16
        old = plsc.load_gather(amt, [pidx])
        upd = candf < old
        plsc.store_scatter(amt, [pidx], jnp.where(upd, candf, old))
        plsc.store_scatter(ax0t, [pidx], x0_v[sl], mask=upd)
        plsc.store_scatter(ax1t, [pidx], x1_v[sl], mask=upd)
        plsc.store_scatter(abt, [pidx], b16, mask=upd)
        return 0

    lax.fori_loop(0, nsteps, body, 0)

    def fold(v, _):
        sl = pl.ds(v * 16, 16)
        am = amt[sl]
        a0 = ax0t[sl]
        a1 = ax1t[sl]
        ab = abt[sl]
        for r in range(1, 16):
            slr = pl.ds(r * _K + v * 16, 16)
            m = amt[slr]
            u = m < am
            am = jnp.where(u, m, am)
            a0 = jnp.where(u, ax0t[slr], a0)
            a1 = jnp.where(u, ax1t[slr], a1)
            ab = jnp.where(u, abt[slr], ab)
        stage[0, sl] = am
        stage[1, sl] = a0
        stage[2, sl] = a1
        stage[3, sl] = ab
        return 0

    lax.fori_loop(0, 16, fold, 0)
    pltpu.sync_copy(stage, bpart_h.at[w])


_sc_b = pl.kernel(
    _sc_b_body,
    out_type=jax.ShapeDtypeStruct((_W, 4, _K), jnp.float32),
    mesh=plsc.VectorSubcoreMesh(core_axis_name="c", subcore_axis_name="s",
                                num_cores=2, num_subcores=16),
    scratch_types=[pltpu.VMEM((_CH,), jnp.int32)]
    + [pltpu.VMEM((_CH,), jnp.float32)] * 3
    + [pltpu.VMEM((_W, 4, _K), jnp.float32),
       pltpu.VMEM((_K,), jnp.float32)]
    + [pltpu.VMEM((16 * _K,), jnp.float32)] * 4
    + [pltpu.VMEM((4, _K), jnp.float32)],
    compiler_params=pltpu.CompilerParams(needs_layout_passes=False),
)


def _tc_q_body(b_ref, q_ref):
    beta = jnp.clip(b_ref[...], 1e-6, 1.0 - 1e-6)
    q_ref[...] = _q_of(beta)


@jax.jit
def _tc_q(beta2d):
    return pl.pallas_call(
        _tc_q_body,
        out_shape=jax.ShapeDtypeStruct(beta2d.shape, jnp.float32),
    )(beta2d)


def _tc_c_body(feat_ref, stats_ref, bpart_ref, noise_ref, out_ref, xa0r, xa1r,
               wrow, smem):
    t = pl.program_id(0)
    iotak = jax.lax.broadcasted_iota(jnp.int32, (1, _K), 1)

    @pl.when(t == 0)
    def _pro():
        s3 = stats_ref[...]
        segc = jnp.sum(s3[:, 0, :], axis=0, keepdims=True)
        betak = jnp.max(s3[:, 1, :], axis=0, keepdims=True)
        paynum = jnp.sum(s3[:, 2, :], axis=0, keepdims=True)
        payden = jnp.sum(s3[:, 3, :], axis=0, keepdims=True)
        b3 = bpart_ref[...]
        am = b3[:, 0, :]
        tm = jnp.min(am, axis=0, keepdims=True)
        sel = (am == tm) & (am < _BIG)
        xa0 = jnp.sum(jnp.where(sel, b3[:, 1, :], 0.0), axis=0, keepdims=True)
        xa1 = jnp.sum(jnp.where(sel, b3[:, 2, :], 0.0), axis=0, keepdims=True)
        ab = jnp.sum(jnp.where(sel, b3[:, 3, :], 0.0), axis=0, keepdims=True)
        validf = ((segc > 0.0) & (iotak > 0)).astype(jnp.float32)
        ba = jnp.clip(ab, 1e-6, 1.0 - 1e-6)
        qa = _q_of(ba)
        xa0r[...] = xa0
        xa1r[...] = xa1
        wrow[...] = qa * validf
        nv = jnp.sum(validf)
        nrow = jnp.sum(noise_ref[...], axis=0, keepdims=True)
        nbsum = jnp.sum(nrow[:, 0:16])
        nnsum = jnp.sum(nrow[:, 16:32])
        noise_l = _SB * nbsum / (nnsum + 1e-9)
        bkv = jnp.clip(betak, 0.0, 1.0)
        beta_obj = jnp.sum(jnp.where(validf > 0.0, 1.0 - bkv, 0.0)) / (
            nv + 1e-9)
        pdz = payden == 0.0
        payk = jnp.where(pdz, 0.0, paynum / jnp.where(pdz, 1.0, payden))
        pay_l = jnp.sum(validf * payk) / (nv + 1e-9)
        smem[0] = 0.0
        smem[1] = 0.0
        smem[2] = noise_l + beta_obj + pay_l
        smem[3] = jnp.sum(segc * validf) + 1e-9
        smem[4] = jnp.float32(_N) * nv + 1e-9
        out_ref[...] = jnp.zeros_like(out_ref)

    feat = feat_ref[...]
    q = feat[:, 0:1]
    x0 = feat[:, 1:2]
    x1 = feat[:, 2:3]
    tidf = feat[:, 3:4]
    onehot = tidf == iotak.astype(jnp.float32)
    d2 = (x0 - xa0r[...]) ** 2 + (x1 - xa1r[...]) ** 2 + 1e-6
    dist = d2 * lax.rsqrt(d2)
    r = jnp.maximum(1.0 - dist, 0.0)
    rq = r * q
    d2q = d2 * q
    attcol = jnp.sum(jnp.where(onehot, d2q, 0.0), axis=0, keepdims=True)
    repcol = jnp.sum(jnp.where(onehot, 0.0, rq), axis=0, keepdims=True)
    smem[0] += jnp.sum(attcol * wrow[...])
    smem[1] += jnp.sum(repcol * wrow[...])

    @pl.when(t == _T - 1)
    def _fin():
        loss = smem[0] / smem[3] + smem[1] / smem[4] + smem[2]
        out_ref[...] = loss * jnp.ones_like(out_ref)


@jax.jit
def _tc_c(feat, stats, bpart, noisep):
    return pl.pallas_call(
        _tc_c_body,
        grid=(_T,),
        in_specs=[
            pl.BlockSpec((_TILE, 4), lambda t: (t, 0)),
            pl.BlockSpec((_W, 4, _K), lambda t: (0, 0, 0)),
            pl.BlockSpec((_W, 4, _K), lambda t: (0, 0, 0)),
            pl.BlockSpec((_W, 128), lambda t: (0, 0)),
        ],
        out_specs=pl.BlockSpec((1, 1), lambda t: (0, 0)),
        out_shape=jax.ShapeDtypeStruct((1, 1), jnp.float32),
        scratch_shapes=[
            pltpu.VMEM((1, _K), jnp.float32),
            pltpu.VMEM((1, _K), jnp.float32),
            pltpu.VMEM((1, _K), jnp.float32),
            pltpu.SMEM((8,), jnp.float32),
        ],
    )(feat, stats, bpart, noisep)


def kernel(pred_beta, pred_ccoords, pred_energy, pred_pos, pred_time,
           pred_id, t_idx, t_energy, t_pos, t_time):
    tid_p = t_idx.reshape(-1).astype(jnp.int32)
    beta_p = pred_beta.reshape(-1)
    x0_p = pred_ccoords[:, 0]
    x1_p = pred_ccoords[:, 1]

    stats, noisep = _sc_a(tid_p, beta_p, pred_energy.reshape(-1),
                          pred_pos[:, 0], pred_pos[:, 1],
                          pred_time.reshape(-1), t_energy.reshape(-1),
                          t_pos[:, 0], t_pos[:, 1], t_time.reshape(-1))
    bpart = _sc_b(tid_p, beta_p, x0_p, x1_p, stats)

    q2d = _tc_q(beta_p.reshape(800, 125))
    q_col = q2d.reshape(_N, 1)
    tidf = t_idx.reshape(-1, 1).astype(jnp.float32)
    feat = jnp.concatenate([q_col, pred_ccoords, tidf], axis=1)
    loss = _tc_c(feat, stats, bpart, noisep)
    return (pred_beta, loss.reshape(1))
